# baseline (speedup 1.0000x reference)
import dataclasses
import functools
import jax
import jax.numpy as jnp
from jax import lax
from jax.experimental import pallas as pl
from jax.experimental.pallas import tpu as pltpu
from jax.experimental.pallas import tpu_sc as plsc

ROW = 8
LANES = 16
W = 512
SLAB = 128


def kernel(s, arclengths, curve_control_points):
    n_seg = curve_control_points.shape[0]
    b = s.shape[0]
    table = curve_control_points.reshape(n_seg, ROW)
    mesh = plsc.VectorSubcoreMesh(core_axis_name="c", subcore_axis_name="s")
    cp = pltpu.CompilerParams()
    if "needs_layout_passes" in pltpu.CompilerParams.__dataclass_fields__:
        cp = dataclasses.replace(cp, needs_layout_passes=False)
    if "use_tc_tiling_on_sc" in pltpu.CompilerParams.__dataclass_fields__:
        cp = dataclasses.replace(cp, use_tc_tiling_on_sc=False)

    @functools.partial(
        pl.kernel,
        mesh=mesh,
        compiler_params=cp,
        out_type=(
            jax.ShapeDtypeStruct((2 * b,), jnp.float32),
            jax.ShapeDtypeStruct((b,), jnp.int32),
        ),
        scratch_types=[
            pltpu.VMEM((W,), jnp.int32),
            pltpu.VMEM((W, ROW), jnp.float32),
            pltpu.SemaphoreType.DMA,
        ],
    )
    def run(s_hbm, table_hbm, pos_hbm, idx_hbm, idxs_v, rows_v, sem):
        def body(s_blk, pos_blk, idx_blk):
            @pl.loop(0, W, step=LANES)
            def _(o):
                sv = s_blk[pl.ds(o, LANES)]
                ii = jnp.minimum(sv.astype(jnp.int32), n_seg - 1)
                ii = jnp.maximum(ii, 0)
                idxs_v[pl.ds(o, LANES)] = ii
                idx_blk[pl.ds(o, LANES)] = ii

            copies = [
                pltpu.async_copy(
                    table_hbm.at[idxs_v.at[pl.ds(k * SLAB, SLAB)]],
                    rows_v.at[pl.ds(k * SLAB, SLAB)],
                    sem,
                )
                for k in range(W // SLAB)
            ]
            for h in copies:
                h.wait()

            @pl.loop(0, W, step=LANES)
            def _(o):
                sv = s_blk[pl.ds(o, LANES)]
                fi = idxs_v[pl.ds(o, LANES)].astype(jnp.float32)
                t = sv - fi
                u = 1.0 - t
                t2 = t * t
                u2 = u * u
                b0 = u2 * u
                b1 = 3.0 * t * u2
                b2 = 3.0 * t2 * u
                b3 = t2 * t
                rid = o + lax.iota(jnp.int32, LANES)
                c = [
                    plsc.load_gather(rows_v, [rid, jnp.full((LANES,), j, jnp.int32)])
                    for j in range(ROW)
                ]
                px = b0 * c[0] + b1 * c[2] + b2 * c[4] + b3 * c[6]
                py = b0 * c[1] + b1 * c[3] + b2 * c[5] + b3 * c[7]
                xid = 2 * rid
                plsc.store_scatter(pos_blk, [xid], px)
                plsc.store_scatter(pos_blk, [xid + 1], py)

        pltpu.emit_pipeline(
            body,
            grid=(b // W,),
            in_specs=[pl.BlockSpec((W,), lambda i: (i,))],
            out_specs=[
                pl.BlockSpec((2 * W,), lambda i: (i,)),
                pl.BlockSpec((W,), lambda i: (i,)),
            ],
            core_axis_name=("c", "s"),
            dimension_semantics=(pltpu.PARALLEL,),
        )(s_hbm, pos_hbm, idx_hbm)

    pos_flat, idx = run(s, table)
    return pos_flat.reshape(b, 2), idx


# speedup vs baseline: 3.9843x; 3.9843x over previous
import dataclasses
import functools
import jax
import jax.numpy as jnp
from jax import lax
from jax.experimental import pallas as pl
from jax.experimental.pallas import tpu as pltpu
from jax.experimental.pallas import tpu_sc as plsc

ROW = 8
LANES = 16
W = 512
SLAB = 128


def kernel(s, arclengths, curve_control_points):
    n_seg = curve_control_points.shape[0]
    b = s.shape[0]
    table = curve_control_points.reshape(n_seg, ROW)
    mesh = plsc.VectorSubcoreMesh(core_axis_name="c", subcore_axis_name="s")
    cp = pltpu.CompilerParams()
    if "needs_layout_passes" in pltpu.CompilerParams.__dataclass_fields__:
        cp = dataclasses.replace(cp, needs_layout_passes=False)
    if "use_tc_tiling_on_sc" in pltpu.CompilerParams.__dataclass_fields__:
        cp = dataclasses.replace(cp, use_tc_tiling_on_sc=False)

    @functools.partial(
        pl.kernel,
        mesh=mesh,
        compiler_params=cp,
        out_type=(
            jax.ShapeDtypeStruct((2, b), jnp.float32),
            jax.ShapeDtypeStruct((b,), jnp.int32),
        ),
        scratch_types=[
            pltpu.VMEM((W,), jnp.int32),
            pltpu.VMEM((W, ROW), jnp.float32),
            pltpu.SemaphoreType.DMA,
        ],
    )
    def run(s_hbm, table_hbm, pos_hbm, idx_hbm, idxs_v, rows_v, sem):
        def body(s_blk, pos_blk, idx_blk):
            @pl.loop(0, W, step=LANES)
            def _(o):
                sv = s_blk[pl.ds(o, LANES)]
                ii = jnp.minimum(sv.astype(jnp.int32), n_seg - 1)
                ii = jnp.maximum(ii, 0)
                idxs_v[pl.ds(o, LANES)] = ii
                idx_blk[pl.ds(o, LANES)] = ii

            copies = [
                pltpu.async_copy(
                    table_hbm.at[idxs_v.at[pl.ds(k * SLAB, SLAB)]],
                    rows_v.at[pl.ds(k * SLAB, SLAB)],
                    sem,
                )
                for k in range(W // SLAB)
            ]
            for h in copies:
                h.wait()

            @pl.loop(0, W, step=LANES)
            def _(o):
                sv = s_blk[pl.ds(o, LANES)]
                fi = idxs_v[pl.ds(o, LANES)].astype(jnp.float32)
                t = sv - fi
                u = 1.0 - t
                t2 = t * t
                u2 = u * u
                b0 = u2 * u
                b1 = 3.0 * t * u2
                b2 = 3.0 * t2 * u
                b3 = t2 * t
                rid = o + lax.iota(jnp.int32, LANES)
                c = [
                    plsc.load_gather(rows_v, [rid, jnp.full((LANES,), j, jnp.int32)])
                    for j in range(ROW)
                ]
                px = b0 * c[0] + b1 * c[2] + b2 * c[4] + b3 * c[6]
                py = b0 * c[1] + b1 * c[3] + b2 * c[5] + b3 * c[7]
                pos_blk[0, pl.ds(o, LANES)] = px
                pos_blk[1, pl.ds(o, LANES)] = py

        pltpu.emit_pipeline(
            body,
            grid=(b // W,),
            in_specs=[pl.BlockSpec((W,), lambda i: (i,))],
            out_specs=[
                pl.BlockSpec((2, W), lambda i: (0, i)),
                pl.BlockSpec((W,), lambda i: (i,)),
            ],
            core_axis_name=("c", "s"),
            dimension_semantics=(pltpu.PARALLEL,),
        )(s_hbm, pos_hbm, idx_hbm)

    pos_planar, idx = run(s, table)
    return pos_planar.T, idx


# speedup vs baseline: 5.1367x; 1.2892x over previous
import dataclasses
import functools
import jax
import jax.numpy as jnp
from jax import lax
from jax.experimental import pallas as pl
from jax.experimental.pallas import tpu as pltpu
from jax.experimental.pallas import tpu_sc as plsc

ROW = 8
LANES = 16
W = 1024
SLAB = 128
NSLAB = W // SLAB
LOOK = 4


def kernel(s, arclengths, curve_control_points):
    n_seg = curve_control_points.shape[0]
    b = s.shape[0]
    table = curve_control_points.reshape(n_seg, ROW)
    mesh = plsc.VectorSubcoreMesh(core_axis_name="c", subcore_axis_name="s")
    cp = pltpu.CompilerParams()
    if "needs_layout_passes" in pltpu.CompilerParams.__dataclass_fields__:
        cp = dataclasses.replace(cp, needs_layout_passes=False)
    if "use_tc_tiling_on_sc" in pltpu.CompilerParams.__dataclass_fields__:
        cp = dataclasses.replace(cp, use_tc_tiling_on_sc=False)

    @functools.partial(
        pl.kernel,
        mesh=mesh,
        compiler_params=cp,
        out_type=(
            jax.ShapeDtypeStruct((2, b), jnp.float32),
            jax.ShapeDtypeStruct((b,), jnp.int32),
        ),
        scratch_types=[
            pltpu.VMEM((W,), jnp.int32),
            pltpu.VMEM((W, ROW), jnp.float32),
        ]
        + [pltpu.SemaphoreType.DMA] * LOOK,
    )
    def run(s_hbm, table_hbm, pos_hbm, idx_hbm, idxs_v, rows_v, *sems):
        def body(s_blk, pos_blk, idx_blk):
            def pass_a(k):
                @pl.loop(k * SLAB, (k + 1) * SLAB, step=LANES)
                def _(o):
                    sv = s_blk[pl.ds(o, LANES)]
                    ii = jnp.minimum(sv.astype(jnp.int32), n_seg - 1)
                    ii = jnp.maximum(ii, 0)
                    idxs_v[pl.ds(o, LANES)] = ii
                    idx_blk[pl.ds(o, LANES)] = ii

            def issue(k):
                return pltpu.async_copy(
                    table_hbm.at[idxs_v.at[pl.ds(k * SLAB, SLAB)]],
                    rows_v.at[pl.ds(k * SLAB, SLAB)],
                    sems[k % LOOK],
                )

            def pass_b(k):
                @pl.loop(k * SLAB, (k + 1) * SLAB, step=LANES)
                def _(o):
                    sv = s_blk[pl.ds(o, LANES)]
                    fi = idxs_v[pl.ds(o, LANES)].astype(jnp.float32)
                    t = sv - fi
                    u = 1.0 - t
                    t2 = t * t
                    u2 = u * u
                    b0 = u2 * u
                    b1 = 3.0 * t * u2
                    b2 = 3.0 * t2 * u
                    b3 = t2 * t
                    rid = o + lax.iota(jnp.int32, LANES)
                    c = [
                        plsc.load_gather(
                            rows_v, [rid, jnp.full((LANES,), j, jnp.int32)]
                        )
                        for j in range(ROW)
                    ]
                    px = b0 * c[0] + b1 * c[2] + b2 * c[4] + b3 * c[6]
                    py = b0 * c[1] + b1 * c[3] + b2 * c[5] + b3 * c[7]
                    pos_blk[0, pl.ds(o, LANES)] = px
                    pos_blk[1, pl.ds(o, LANES)] = py

            handles = [None] * NSLAB
            for k in range(NSLAB):
                pass_a(k)
                if k >= LOOK:
                    handles[k - LOOK].wait()
                handles[k] = issue(k)
                if k >= LOOK:
                    pass_b(k - LOOK)
            for k in range(NSLAB - LOOK, NSLAB):
                handles[k].wait()
                pass_b(k)

        pltpu.emit_pipeline(
            body,
            grid=(b // W,),
            in_specs=[pl.BlockSpec((W,), lambda i: (i,))],
            out_specs=[
                pl.BlockSpec((2, W), lambda i: (0, i)),
                pl.BlockSpec((W,), lambda i: (i,)),
            ],
            core_axis_name=("c", "s"),
            dimension_semantics=(pltpu.PARALLEL,),
        )(s_hbm, pos_hbm, idx_hbm)

    pos_planar, idx = run(s, table)
    return pos_planar.T, idx


# speedup vs baseline: 5.3884x; 1.0490x over previous
import dataclasses
import functools
import jax
import jax.numpy as jnp
from jax import lax
from jax.experimental import pallas as pl
from jax.experimental.pallas import tpu as pltpu
from jax.experimental.pallas import tpu_sc as plsc

ROW = 8
LANES = 16
W = 1024
SLAB = 128
NSLAB = W // SLAB
LOOK = 4


def kernel(s, arclengths, curve_control_points):
    n_seg = curve_control_points.shape[0]
    b = s.shape[0]
    table = curve_control_points.reshape(n_seg, ROW)
    mesh = plsc.VectorSubcoreMesh(core_axis_name="c", subcore_axis_name="s")
    cp = pltpu.CompilerParams()
    if "needs_layout_passes" in pltpu.CompilerParams.__dataclass_fields__:
        cp = dataclasses.replace(cp, needs_layout_passes=False)
    if "use_tc_tiling_on_sc" in pltpu.CompilerParams.__dataclass_fields__:
        cp = dataclasses.replace(cp, use_tc_tiling_on_sc=False)

    @functools.partial(
        pl.kernel,
        mesh=mesh,
        compiler_params=cp,
        out_type=(
            jax.ShapeDtypeStruct((2, b), jnp.float32),
            jax.ShapeDtypeStruct((b,), jnp.int32),
        ),
        scratch_types=[
            pltpu.VMEM((W,), jnp.int32),
            pltpu.VMEM((W, ROW), jnp.float32),
        ]
        + [pltpu.SemaphoreType.DMA] * (LOOK + 1),
    )
    def run(s_hbm, table_hbm, pos_hbm, idx_hbm, idxs_v, rows_v, *sems):
        def body(s_blk, pos_blk, idx_blk):
            def pass_a(k):
                @plsc.parallel_loop(k * SLAB, (k + 1) * SLAB, step=LANES, unroll=2)
                def _(o):
                    sv = s_blk[pl.ds(o, LANES)]
                    ii = jnp.minimum(sv.astype(jnp.int32), n_seg - 1)
                    ii = jnp.maximum(ii, 0)
                    idxs_v[pl.ds(o, LANES)] = ii
                    idx_blk[pl.ds(o, LANES)] = ii

            def issue(k):
                return pltpu.async_copy(
                    table_hbm.at[idxs_v.at[pl.ds(k * SLAB, SLAB)]],
                    rows_v.at[pl.ds(k * SLAB, SLAB)],
                    sems[k % LOOK],
                )

            def pass_b(k):
                @plsc.parallel_loop(k * SLAB, (k + 1) * SLAB, step=LANES, unroll=2)
                def _(o):
                    sv = s_blk[pl.ds(o, LANES)]
                    fi = idxs_v[pl.ds(o, LANES)].astype(jnp.float32)
                    t = sv - fi
                    u = 1.0 - t
                    t2 = t * t
                    u2 = u * u
                    b0 = u2 * u
                    b1 = 3.0 * t * u2
                    b2 = 3.0 * t2 * u
                    b3 = t2 * t
                    rid = o + lax.iota(jnp.int32, LANES)
                    c = [
                        plsc.load_gather(
                            rows_v, [rid, jnp.full((LANES,), j, jnp.int32)]
                        )
                        for j in range(ROW)
                    ]
                    px = b0 * c[0] + b1 * c[2] + b2 * c[4] + b3 * c[6]
                    py = b0 * c[1] + b1 * c[3] + b2 * c[5] + b3 * c[7]
                    pos_blk[0, pl.ds(o, LANES)] = px
                    pos_blk[1, pl.ds(o, LANES)] = py

            handles = [None] * NSLAB
            for k in range(NSLAB):
                pass_a(k)
                if k >= LOOK:
                    handles[k - LOOK].wait()
                handles[k] = issue(k)
                if k >= LOOK:
                    pass_b(k - LOOK)
            for k in range(NSLAB - LOOK, NSLAB):
                handles[k].wait()
                pass_b(k)

        pltpu.emit_pipeline(
            body,
            grid=(b // W,),
            in_specs=[pl.BlockSpec((W,), lambda i: (i,))],
            out_specs=[
                pl.BlockSpec((2, W), lambda i: (0, i)),
                pl.BlockSpec((W,), lambda i: (i,)),
            ],
            core_axis_name=("c", "s"),
            dimension_semantics=(pltpu.PARALLEL,),
        )(s_hbm, pos_hbm, idx_hbm)

    pos_planar, idx = run(s, table)
    return pos_planar.T, idx


# speedup vs baseline: 5.6296x; 1.0448x over previous
import dataclasses
import functools
import jax
import jax.numpy as jnp
from jax import lax
from jax.experimental import pallas as pl
from jax.experimental.pallas import tpu as pltpu
from jax.experimental.pallas import tpu_sc as plsc

ROW = 8
LANES = 16
W = 2048
SLAB = 128
NSLAB = W // SLAB
LOOK = 6


def kernel(s, arclengths, curve_control_points):
    n_seg = curve_control_points.shape[0]
    b = s.shape[0]
    table = curve_control_points.reshape(n_seg, ROW)
    mesh = plsc.VectorSubcoreMesh(core_axis_name="c", subcore_axis_name="s")
    cp = pltpu.CompilerParams()
    if "needs_layout_passes" in pltpu.CompilerParams.__dataclass_fields__:
        cp = dataclasses.replace(cp, needs_layout_passes=False)
    if "use_tc_tiling_on_sc" in pltpu.CompilerParams.__dataclass_fields__:
        cp = dataclasses.replace(cp, use_tc_tiling_on_sc=False)

    @functools.partial(
        pl.kernel,
        mesh=mesh,
        compiler_params=cp,
        out_type=(
            jax.ShapeDtypeStruct((2, b), jnp.float32),
            jax.ShapeDtypeStruct((b,), jnp.int32),
        ),
        scratch_types=[
            pltpu.VMEM((W,), jnp.int32),
            pltpu.VMEM((W, ROW), jnp.float32),
        ]
        + [pltpu.SemaphoreType.DMA] * (LOOK + 1),
    )
    def run(s_hbm, table_hbm, pos_hbm, idx_hbm, idxs_v, rows_v, *sems):
        def body(s_blk, pos_blk, idx_blk):
            def pass_a(k):
                @plsc.parallel_loop(k * SLAB, (k + 1) * SLAB, step=LANES, unroll=2)
                def _(o):
                    sv = s_blk[pl.ds(o, LANES)]
                    ii = jnp.minimum(sv.astype(jnp.int32), n_seg - 1)
                    ii = jnp.maximum(ii, 0)
                    idxs_v[pl.ds(o, LANES)] = ii
                    idx_blk[pl.ds(o, LANES)] = ii

            def issue(k):
                return pltpu.async_copy(
                    table_hbm.at[idxs_v.at[pl.ds(k * SLAB, SLAB)]],
                    rows_v.at[pl.ds(k * SLAB, SLAB)],
                    sems[k % LOOK],
                )

            def pass_b(k):
                @plsc.parallel_loop(k * SLAB, (k + 1) * SLAB, step=LANES, unroll=2)
                def _(o):
                    sv = s_blk[pl.ds(o, LANES)]
                    fi = idxs_v[pl.ds(o, LANES)].astype(jnp.float32)
                    t = sv - fi
                    u = 1.0 - t
                    t2 = t * t
                    u2 = u * u
                    b0 = u2 * u
                    b1 = 3.0 * t * u2
                    b2 = 3.0 * t2 * u
                    b3 = t2 * t
                    rid = o + lax.iota(jnp.int32, LANES)
                    c = [
                        plsc.load_gather(
                            rows_v, [rid, jnp.full((LANES,), j, jnp.int32)]
                        )
                        for j in range(ROW)
                    ]
                    px = b0 * c[0] + b1 * c[2] + b2 * c[4] + b3 * c[6]
                    py = b0 * c[1] + b1 * c[3] + b2 * c[5] + b3 * c[7]
                    pos_blk[0, pl.ds(o, LANES)] = px
                    pos_blk[1, pl.ds(o, LANES)] = py

            handles = [None] * NSLAB
            for k in range(NSLAB):
                pass_a(k)
                if k >= LOOK:
                    handles[k - LOOK].wait()
                handles[k] = issue(k)
                if k >= LOOK:
                    pass_b(k - LOOK)
            for k in range(NSLAB - LOOK, NSLAB):
                handles[k].wait()
                pass_b(k)

        pltpu.emit_pipeline(
            body,
            grid=(b // W,),
            in_specs=[pl.BlockSpec((W,), lambda i: (i,))],
            out_specs=[
                pl.BlockSpec((2, W), lambda i: (0, i)),
                pl.BlockSpec((W,), lambda i: (i,)),
            ],
            core_axis_name=("c", "s"),
            dimension_semantics=(pltpu.PARALLEL,),
        )(s_hbm, pos_hbm, idx_hbm)

    pos_planar, idx = run(s, table)
    return pos_planar.T, idx


# speedup vs baseline: 6.1913x; 1.0998x over previous
import dataclasses
import functools
import jax
import jax.numpy as jnp
from jax import lax
from jax.experimental import pallas as pl
from jax.experimental.pallas import tpu as pltpu
from jax.experimental.pallas import tpu_sc as plsc

ROW = 8
LANES = 16
W = 2048
SLAB = 256
NSLAB = W // SLAB
LOOK = 4


def kernel(s, arclengths, curve_control_points):
    n_seg = curve_control_points.shape[0]
    b = s.shape[0]
    table = curve_control_points.reshape(n_seg, ROW)
    mesh = plsc.VectorSubcoreMesh(core_axis_name="c", subcore_axis_name="s")
    cp = pltpu.CompilerParams()
    if "needs_layout_passes" in pltpu.CompilerParams.__dataclass_fields__:
        cp = dataclasses.replace(cp, needs_layout_passes=False)
    if "use_tc_tiling_on_sc" in pltpu.CompilerParams.__dataclass_fields__:
        cp = dataclasses.replace(cp, use_tc_tiling_on_sc=False)

    @functools.partial(
        pl.kernel,
        mesh=mesh,
        compiler_params=cp,
        out_type=(
            jax.ShapeDtypeStruct((2, b), jnp.float32),
            jax.ShapeDtypeStruct((b,), jnp.int32),
        ),
        scratch_types=[
            pltpu.VMEM((W,), jnp.int32),
            pltpu.VMEM((W, ROW), jnp.float32),
        ]
        + [pltpu.SemaphoreType.DMA] * (LOOK + 1),
    )
    def run(s_hbm, table_hbm, pos_hbm, idx_hbm, idxs_v, rows_v, *sems):
        def body(s_blk, pos_blk, idx_blk):
            def pass_a(k):
                @plsc.parallel_loop(k * SLAB, (k + 1) * SLAB, step=LANES, unroll=2)
                def _(o):
                    sv = s_blk[pl.ds(o, LANES)]
                    ii = jnp.minimum(sv.astype(jnp.int32), n_seg - 1)
                    ii = jnp.maximum(ii, 0)
                    idxs_v[pl.ds(o, LANES)] = ii
                    idx_blk[pl.ds(o, LANES)] = ii

            def issue(k):
                return pltpu.async_copy(
                    table_hbm.at[idxs_v.at[pl.ds(k * SLAB, SLAB)]],
                    rows_v.at[pl.ds(k * SLAB, SLAB)],
                    sems[k % LOOK],
                )

            def pass_b(k):
                @plsc.parallel_loop(k * SLAB, (k + 1) * SLAB, step=LANES, unroll=2)
                def _(o):
                    sv = s_blk[pl.ds(o, LANES)]
                    fi = idxs_v[pl.ds(o, LANES)].astype(jnp.float32)
                    t = sv - fi
                    u = 1.0 - t
                    t2 = t * t
                    u2 = u * u
                    b0 = u2 * u
                    b1 = 3.0 * t * u2
                    b2 = 3.0 * t2 * u
                    b3 = t2 * t
                    rid = o + lax.iota(jnp.int32, LANES)
                    c = [
                        plsc.load_gather(
                            rows_v, [rid, jnp.full((LANES,), j, jnp.int32)]
                        )
                        for j in range(ROW)
                    ]
                    px = b0 * c[0] + b1 * c[2] + b2 * c[4] + b3 * c[6]
                    py = b0 * c[1] + b1 * c[3] + b2 * c[5] + b3 * c[7]
                    pos_blk[0, pl.ds(o, LANES)] = px
                    pos_blk[1, pl.ds(o, LANES)] = py

            handles = [None] * NSLAB
            for k in range(NSLAB):
                pass_a(k)
                if k >= LOOK:
                    handles[k - LOOK].wait()
                handles[k] = issue(k)
                if k >= LOOK:
                    pass_b(k - LOOK)
            for k in range(NSLAB - LOOK, NSLAB):
                handles[k].wait()
                pass_b(k)

        pltpu.emit_pipeline(
            body,
            grid=(b // W,),
            in_specs=[pl.BlockSpec((W,), lambda i: (i,))],
            out_specs=[
                pl.BlockSpec((2, W), lambda i: (0, i)),
                pl.BlockSpec((W,), lambda i: (i,)),
            ],
            core_axis_name=("c", "s"),
            dimension_semantics=(pltpu.PARALLEL,),
        )(s_hbm, pos_hbm, idx_hbm)

    pos_planar, idx = run(s, table)
    return pos_planar.T, idx
